# baseline (speedup 1.0000x reference)
import jax
import jax.numpy as jnp
from jax import lax
from jax.experimental import pallas as pl
from jax.experimental.pallas import tpu as pltpu
from jax.experimental.pallas import tpu_sc as plsc

NUM_ROWS = 128
NUM_COLS = 32768
LANES = 16
CHUNKS = NUM_COLS // LANES
NUM_WORKERS = 32
ROWS_PER_WORKER = NUM_ROWS // NUM_WORKERS
PATCH = 128
U = 8
INT_MAX = 2**31 - 1


def _argmax_row(xrow, lane_iota):
    def scan_body(i, accs):
        out = []
        base = i * (U * LANES)
        bi = jnp.full((LANES,), 0, jnp.int32) + i
        for u in range(U):
            vmax, viter = accs[2 * u], accs[2 * u + 1]
            v = xrow[pl.ds(base + u * LANES, LANES)]
            cond = v > vmax
            out.append(jnp.where(cond, v, vmax))
            out.append(jnp.where(cond, bi, viter))
        return tuple(out)

    init = []
    for _ in range(U):
        init.append(jnp.full((LANES,), -jnp.inf, jnp.float32))
        init.append(jnp.zeros((LANES,), jnp.int32))
    accs = lax.fori_loop(0, CHUNKS // U, scan_body, tuple(init))

    best_v = accs[0]
    best_i = accs[1] * (U * LANES) + lane_iota
    for u in range(1, U):
        v = accs[2 * u]
        idx = accs[2 * u + 1] * (U * LANES) + (u * LANES) + lane_iota
        better = (v > best_v) | ((v == best_v) & (idx < best_i))
        best_v = jnp.where(better, v, best_v)
        best_i = jnp.where(better, idx, best_i)

    gmax = jnp.max(best_v)
    cand = jnp.where(best_v == gmax, best_i, jnp.int32(INT_MAX))
    return jnp.min(cand)


def _body(x_hbm, out_hbm, xbuf0, xbuf1, zeros_buf, patch_buf, sem_in,
          sem_zero, sem_patch):
    xbufs = [xbuf0, xbuf1]
    wid = lax.axis_index("s") * 2 + lax.axis_index("c")
    row0 = wid * ROWS_PER_WORKER

    in_copy = pltpu.async_copy(x_hbm.at[row0], xbuf0, sem_in)

    @plsc.parallel_loop(0, CHUNKS, unroll=U)
    def _zero(i):
        zeros_buf[pl.ds(i * LANES, LANES)] = jnp.zeros((LANES,), jnp.int32)

    zero_copies = []
    for r in range(ROWS_PER_WORKER):
        zero_copies.append(
            pltpu.async_copy(zeros_buf, out_hbm.at[row0 + r], sem_zero))

    lane_iota = lax.broadcasted_iota(jnp.int32, (LANES,), 0)
    bases = []

    for r in range(ROWS_PER_WORKER):
        in_copy.wait()
        if r + 1 < ROWS_PER_WORKER:
            in_copy = pltpu.async_copy(
                x_hbm.at[row0 + r + 1], xbufs[(r + 1) % 2], sem_in)

        idx = _argmax_row(xbufs[r % 2], lane_iota)

        base = pl.multiple_of(jnp.int32(-PATCH) & idx, PATCH)
        off = idx - base
        for j in range(PATCH // LANES):
            patch_buf[pl.ds(r * PATCH + j * LANES, LANES)] = (
                lane_iota == (off - j * LANES)).astype(jnp.int32)
        bases.append(base)

    for c in zero_copies:
        c.wait()
    patch_copies = []
    for r in range(ROWS_PER_WORKER):
        patch_copies.append(pltpu.async_copy(
            patch_buf.at[pl.ds(r * PATCH, PATCH)],
            out_hbm.at[row0 + r].at[pl.ds(bases[r], PATCH)],
            sem_patch))
    for c in patch_copies:
        c.wait()


@jax.jit
def _hardmax_sc(x):
    mesh = plsc.VectorSubcoreMesh(core_axis_name="c", subcore_axis_name="s",
                                  num_cores=2, num_subcores=16)
    return pl.kernel(
        _body,
        out_type=jax.ShapeDtypeStruct((NUM_ROWS, NUM_COLS), jnp.int32),
        mesh=mesh,
        scratch_types=[
            pltpu.VMEM((NUM_COLS,), jnp.float32),
            pltpu.VMEM((NUM_COLS,), jnp.float32),
            pltpu.VMEM((NUM_COLS,), jnp.int32),
            pltpu.VMEM((ROWS_PER_WORKER * PATCH,), jnp.int32),
            pltpu.SemaphoreType.DMA,
            pltpu.SemaphoreType.DMA,
            pltpu.SemaphoreType.DMA,
        ],
        compiler_params=pltpu.CompilerParams(
            needs_layout_passes=False,
            disable_bounds_checks=True,
            disable_semaphore_checks=True,
            skip_device_barrier=True,
        ),
    )(x)


def kernel(x):
    return _hardmax_sc(x)


# speedup vs baseline: 1.0310x; 1.0310x over previous
import jax
import jax.numpy as jnp
from jax import lax
from jax.experimental import pallas as pl
from jax.experimental.pallas import tpu as pltpu
from jax.experimental.pallas import tpu_sc as plsc

NUM_ROWS = 128
NUM_COLS = 32768
LANES = 16
CHUNKS = NUM_COLS // LANES
NUM_WORKERS = 32
ROWS_PER_WORKER = NUM_ROWS // NUM_WORKERS
PATCH = 128
U = 8
INT_MAX = 2**31 - 1


def _argmax_row(xrow, lane_iota):
    def scan_body(i, accs):
        out = []
        base = i * (U * LANES)
        bi = jnp.full((LANES,), 0, jnp.int32) + i
        for u in range(U):
            vmax, viter = accs[2 * u], accs[2 * u + 1]
            v = xrow[pl.ds(base + u * LANES, LANES)]
            cond = v > vmax
            out.append(jnp.where(cond, v, vmax))
            out.append(jnp.where(cond, bi, viter))
        return tuple(out)

    init = []
    for _ in range(U):
        init.append(jnp.full((LANES,), -jnp.inf, jnp.float32))
        init.append(jnp.zeros((LANES,), jnp.int32))
    accs = lax.fori_loop(0, CHUNKS // U, scan_body, tuple(init))

    best_v = accs[0]
    best_i = accs[1] * (U * LANES) + lane_iota
    for u in range(1, U):
        v = accs[2 * u]
        idx = accs[2 * u + 1] * (U * LANES) + (u * LANES) + lane_iota
        better = (v > best_v) | ((v == best_v) & (idx < best_i))
        best_v = jnp.where(better, v, best_v)
        best_i = jnp.where(better, idx, best_i)

    gmax = jnp.max(best_v)
    cand = jnp.where(best_v == gmax, best_i, jnp.int32(INT_MAX))
    return jnp.min(cand)


def _body(x_hbm, out_hbm, xbuf0, xbuf1, zeros_buf, patch_buf, sem_in,
          sem_zero, sem_patch):
    xbufs = [xbuf0, xbuf1]
    wid = lax.axis_index("s") * 2 + lax.axis_index("c")
    row0 = wid * ROWS_PER_WORKER

    in_copies = [pltpu.async_copy(x_hbm.at[row0], xbuf0, sem_in),
                 pltpu.async_copy(x_hbm.at[row0 + 1], xbuf1, sem_in)]

    @plsc.parallel_loop(0, CHUNKS, unroll=U)
    def _zero(i):
        zeros_buf[pl.ds(i * LANES, LANES)] = jnp.zeros((LANES,), jnp.int32)

    zero_copies = []
    for r in range(ROWS_PER_WORKER):
        zero_copies.append(
            pltpu.async_copy(zeros_buf, out_hbm.at[row0 + r], sem_zero))

    lane_iota = lax.broadcasted_iota(jnp.int32, (LANES,), 0)
    bases = []

    for r in range(ROWS_PER_WORKER):
        in_copies[r % 2].wait()

        idx = _argmax_row(xbufs[r % 2], lane_iota)

        if r + 2 < ROWS_PER_WORKER:
            in_copies[r % 2] = pltpu.async_copy(
                x_hbm.at[row0 + r + 2], xbufs[r % 2], sem_in)

        base = pl.multiple_of(jnp.int32(-PATCH) & idx, PATCH)
        off = idx - base
        for j in range(PATCH // LANES):
            patch_buf[pl.ds(r * PATCH + j * LANES, LANES)] = (
                lane_iota == (off - j * LANES)).astype(jnp.int32)
        bases.append(base)

    for c in zero_copies:
        c.wait()
    patch_copies = []
    for r in range(ROWS_PER_WORKER):
        patch_copies.append(pltpu.async_copy(
            patch_buf.at[pl.ds(r * PATCH, PATCH)],
            out_hbm.at[row0 + r].at[pl.ds(bases[r], PATCH)],
            sem_patch))
    for c in patch_copies:
        c.wait()


@jax.jit
def _hardmax_sc(x):
    mesh = plsc.VectorSubcoreMesh(core_axis_name="c", subcore_axis_name="s",
                                  num_cores=2, num_subcores=16)
    return pl.kernel(
        _body,
        out_type=jax.ShapeDtypeStruct((NUM_ROWS, NUM_COLS), jnp.int32),
        mesh=mesh,
        scratch_types=[
            pltpu.VMEM((NUM_COLS,), jnp.float32),
            pltpu.VMEM((NUM_COLS,), jnp.float32),
            pltpu.VMEM((NUM_COLS,), jnp.int32),
            pltpu.VMEM((ROWS_PER_WORKER * PATCH,), jnp.int32),
            pltpu.SemaphoreType.DMA,
            pltpu.SemaphoreType.DMA,
            pltpu.SemaphoreType.DMA,
        ],
        compiler_params=pltpu.CompilerParams(needs_layout_passes=False),
    )(x)


def kernel(x):
    return _hardmax_sc(x)
